# baseline (speedup 1.0000x reference)
import functools

import jax
import jax.numpy as jnp
from jax import lax
from jax.experimental import pallas as pl
from jax.experimental.pallas import tpu as pltpu
from jax.experimental.pallas import tpu_sc as plsc

S = 8192
D = 128
LOW = 1024
CH = 512
NMX = 0
HP = 4
H = 32


def _rank_body(q_ref, k_hbm, slots_ref, kbuf, sem):
    i = pl.program_id(0)
    nsteps = H // HP
    slot = lax.rem(i, 2)
    nslot = lax.rem(i + 1, 2)

    @pl.when(i == 0)
    def _():
        h0 = i * HP
        pltpu.make_async_copy(
            k_hbm.at[h0 // 16, pl.ds(h0 % 16, HP)], kbuf.at[slot],
            sem.at[slot]).start()

    @pl.when(i + 1 < nsteps)
    def _():
        nh0 = (i + 1) * HP
        pltpu.make_async_copy(
            k_hbm.at[nh0 // 16, pl.ds(nh0 % 16, HP)], kbuf.at[nslot],
            sem.at[nslot]).start()

    h0 = i * HP
    pltpu.make_async_copy(
        k_hbm.at[h0 // 16, pl.ds(h0 % 16, HP)], kbuf.at[slot],
        sem.at[slot]).wait()

    for hh in range(HP):
        _rank_one_head(q_ref[0, hh, 0:1, :], kbuf[slot, hh], slots_ref, hh)


def _rank_one_head(q2, kmat, slots_ref, hh):
    srow = lax.dot_general(q2, kmat, (((1,), (1,)), ((), ())),
                           preferred_element_type=jnp.float32)
    svec = srow.reshape(S)
    a = jnp.concatenate(
        [lax.slice(svec, (1,), (S,)), jnp.full((1,), jnp.inf, jnp.float32)])
    low = lax.slice(a, (0,), (LOW,))
    low_col = low[:, None]
    p_iota = lax.broadcasted_iota(jnp.int32, (LOW, CH), 0)
    i_iota = lax.broadcasted_iota(jnp.int32, (LOW, CH), 1)
    ones_col = jnp.ones((CH, 1), jnp.float32)
    acc = jnp.zeros((LOW, CH), jnp.int32)
    cnt = jnp.zeros((LOW, 1), jnp.float32)
    for c in range(S // CH):
        chunk = lax.slice(a, (c * CH,), ((c + 1) * CH,))[None, :]
        lt = chunk < low_col
        if c < (S // CH) - NMX:
            acc = acc + lt.astype(jnp.int32)
            if c * CH < LOW:
                tie = (chunk == low_col) & ((i_iota + c * CH) < p_iota)
                acc = acc + tie.astype(jnp.int32)
        else:
            m = lt.astype(jnp.float32)
            cnt = cnt + lax.dot_general(m, ones_col, (((1,), (0,)), ((), ())),
                                        preferred_element_type=jnp.float32)
    slots = jnp.sum(acc, axis=1) + cnt.reshape(LOW).astype(jnp.int32) + 1
    slots_ref[0, hh] = slots.reshape(8, 128)


def _rank_call(q, k):
    return pl.pallas_call(
        _rank_body,
        grid=(H // HP,),
        in_specs=[
            pl.BlockSpec((1, HP, 8, D),
                         lambda i: (i // (16 // HP), i % (16 // HP), 0, 0)),
            pl.BlockSpec(memory_space=pl.ANY),
        ],
        out_specs=pl.BlockSpec((1, HP, 8, 128),
                               lambda i: (i // (16 // HP), i % (16 // HP), 0, 0)),
        out_shape=jax.ShapeDtypeStruct((2, 16, 8, 128), jnp.int32),
        scratch_shapes=[
            pltpu.VMEM((2, HP, S, D), jnp.float32),
            pltpu.SemaphoreType.DMA((2,)),
        ],
        compiler_params=pltpu.CompilerParams(
            dimension_semantics=("arbitrary",),
            vmem_limit_bytes=100 * 1024 * 1024,
        ),
    )(q, k)


def _scatter_body(slots_hbm, out_hbm, slots_v, buf_v):
    wid = lax.axis_index("s") * 2 + lax.axis_index("c")
    pltpu.sync_copy(slots_hbm.at[wid], slots_v)
    zero = jnp.zeros((16,), jnp.int32)
    one = jnp.ones((16,), jnp.int32)

    def zbody(i, carry):
        buf_v[pl.ds(pl.multiple_of(i * 16, 16), 16)] = zero
        return carry

    lax.fori_loop(0, S // 16, zbody, 0)

    def sbody(t, carry):
        idx = slots_v[pl.ds(pl.multiple_of(t * 16, 16), 16)]
        plsc.store_scatter(buf_v, [idx], one)
        return carry

    lax.fori_loop(0, LOW // 16, sbody, 0)
    head = buf_v[pl.ds(0, 16)]
    buf_v[pl.ds(0, 16)] = jnp.where(lax.iota(jnp.int32, 16) == 0, 1, head)
    pltpu.sync_copy(buf_v, out_hbm.at[wid])


@functools.cache
def _scatter_call():
    return pl.kernel(
        _scatter_body,
        mesh=plsc.VectorSubcoreMesh(core_axis_name="c", subcore_axis_name="s"),
        out_type=jax.ShapeDtypeStruct((H, S), jnp.int32),
        scratch_types=[
            pltpu.VMEM((LOW,), jnp.int32),
            pltpu.VMEM((S,), jnp.int32),
        ],
        compiler_params=pltpu.CompilerParams(needs_layout_passes=False),
    )


def kernel(q, k):
    slots = _rank_call(q, k).reshape(H, LOW)
    out32 = _scatter_call()(slots)
    return (out32 != 0).reshape(2, 16, S)
